# baseline (speedup 1.0000x reference)
import jax
import jax.numpy as jnp
from jax.experimental import pallas as pl

_G = 32


def _lrelu(v):
    return jnp.maximum(v, 0.2 * v)


def _gat_fused(x_ref, w1_ref, a1s_ref, a1d_ref, b1_ref, gamma_ref, beta_ref,
               w2_ref, a2s_ref, a2d_ref, b2_ref, out_ref):
    g, n, din = x_ref.shape
    dh = w1_ref.shape[1]

    x = x_ref[...].astype(jnp.float32).reshape(g * n, din)
    h_flat = jnp.dot(x, w1_ref[...], preferred_element_type=jnp.float32)
    h = h_flat.reshape(g, n, dh)

    s = (h * a1s_ref[...]).sum(axis=2, keepdims=True)
    d = (h * a1d_ref[...]).sum(axis=2, keepdims=True)
    d_row = jnp.swapaxes(d, 1, 2)
    dmax = jnp.max(d_row, axis=2, keepdims=True)
    m = _lrelu(s + dmax)
    p = jnp.exp(_lrelu(s + d_row) - m)
    ones = jnp.ones((g, n, 1), jnp.float32)
    ho = jnp.concatenate([h, ones], axis=2)
    acc = jax.lax.dot_general(p, ho,
                              (((2,), (1,)), ((0,), (0,))),
                              preferred_element_type=jnp.float32)
    out1 = acc[:, :, :dh] / acc[:, :, dh:] + b1_ref[...]

    mu = out1.mean(axis=2, keepdims=True)
    cent = out1 - mu
    var = (cent * cent).mean(axis=2, keepdims=True)
    xn = cent * jax.lax.rsqrt(var + 1e-5) * gamma_ref[...] + beta_ref[...]

    h2 = (xn * w2_ref[...]).sum(axis=2, keepdims=True)
    sa = a2s_ref[0, 0] * h2
    da = a2d_ref[0, 0] * h2
    da_row = jnp.swapaxes(da, 1, 2)
    m2 = _lrelu(sa + jnp.max(da_row, axis=2, keepdims=True))
    p2 = jnp.exp(_lrelu(sa + da_row) - m2)
    h2o = jnp.concatenate([h2, ones], axis=2)
    acc2 = jax.lax.dot_general(p2, h2o,
                               (((2,), (1,)), ((0,), (0,))),
                               preferred_element_type=jnp.float32)
    out2 = acc2[:, :, 0:1] / acc2[:, :, 1:2] + b2_ref[0, 0]
    out_ref[...] = jnp.where(out2 > 0.0, out2, jnp.exp(out2) - 1.0)


def kernel(x, adj, W1, a1_src, a1_dst, b1, gamma, beta, W2, a2_src, a2_dst, b2):
    del adj
    b, n, din = x.shape
    dh = W1.shape[1]
    full = lambda shape: pl.BlockSpec(shape, lambda i: (0, 0))
    out = pl.pallas_call(
        _gat_fused,
        grid=(b // _G,),
        in_specs=[
            pl.BlockSpec((_G, n, din), lambda i: (i, 0, 0)),
            full((din, dh)),
            full((1, dh)), full((1, dh)), full((1, dh)),
            full((1, dh)), full((1, dh)),
            full((1, dh)),
            full((1, 1)), full((1, 1)), full((1, 1)),
        ],
        out_specs=pl.BlockSpec((_G, n, 1), lambda i: (i, 0, 0)),
        out_shape=jax.ShapeDtypeStruct((b, n, 1), jnp.float32),
    )(x.astype(jnp.bfloat16), W1,
      a1_src.reshape(1, dh), a1_dst.reshape(1, dh), b1.reshape(1, dh),
      gamma.reshape(1, dh), beta.reshape(1, dh),
      W2.reshape(1, dh),
      a2_src.reshape(1, 1), a2_dst.reshape(1, 1), b2.reshape(1, 1))
    return out


# speedup vs baseline: 1.0678x; 1.0678x over previous
import jax
import jax.numpy as jnp
from jax.experimental import pallas as pl

_G = 32


def _lrelu(v):
    return jnp.maximum(v, 0.2 * v)


def _gat_fused(x_ref, w1_ref, a1s_ref, a1d_ref, b1_ref, gamma_ref, beta_ref,
               w2_ref, a2s_ref, a2d_ref, b2_ref, out_ref):
    g, n, din = x_ref.shape
    dh = w1_ref.shape[1]

    x = x_ref[...].astype(jnp.float32).reshape(g * n, din)
    h_flat = jnp.dot(x, w1_ref[...], preferred_element_type=jnp.float32)
    h = h_flat.reshape(g, n, dh)

    s = (h_flat * a1s_ref[...]).sum(axis=1, keepdims=True).reshape(g, n, 1)
    d = (h_flat * a1d_ref[...]).sum(axis=1, keepdims=True).reshape(g, n, 1)
    d_row = jnp.swapaxes(d, 1, 2)
    dmax = jnp.max(d_row, axis=2, keepdims=True)
    m = _lrelu(s + dmax)
    p = jnp.exp(_lrelu(s + d_row) - m)
    ones = jnp.ones((g, n, 1), jnp.bfloat16)
    ho = jnp.concatenate([h.astype(jnp.bfloat16), ones], axis=2)
    acc = jax.lax.dot_general(p.astype(jnp.bfloat16), ho,
                              (((2,), (1,)), ((0,), (0,))),
                              preferred_element_type=jnp.float32)
    out1 = acc[:, :, :dh] / acc[:, :, dh:] + b1_ref[...]

    mu = out1.mean(axis=2, keepdims=True)
    cent = out1 - mu
    var = (cent * cent).mean(axis=2, keepdims=True)
    xn = cent * jax.lax.rsqrt(var + 1e-5) * gamma_ref[...] + beta_ref[...]

    h2 = (xn * w2_ref[...]).sum(axis=2, keepdims=True)
    sa = a2s_ref[0, 0] * h2
    da = a2d_ref[0, 0] * h2
    da_row = jnp.swapaxes(da, 1, 2)
    m2 = _lrelu(sa + jnp.max(da_row, axis=2, keepdims=True))
    p2 = jnp.exp(_lrelu(sa + da_row) - m2)
    h2o = jnp.concatenate([h2.astype(jnp.bfloat16), ones], axis=2)
    acc2 = jax.lax.dot_general(p2.astype(jnp.bfloat16), h2o,
                               (((2,), (1,)), ((0,), (0,))),
                               preferred_element_type=jnp.float32)
    out2 = acc2[:, :, 0] / acc2[:, :, 1] + b2_ref[0, 0]
    out_ref[...] = jnp.where(out2 > 0.0, out2, jnp.exp(out2) - 1.0)


def kernel(x, adj, W1, a1_src, a1_dst, b1, gamma, beta, W2, a2_src, a2_dst, b2):
    del adj
    b, n, din = x.shape
    dh = W1.shape[1]
    full = lambda shape: pl.BlockSpec(shape, lambda i: (0, 0))
    out = pl.pallas_call(
        _gat_fused,
        grid=(b // _G,),
        in_specs=[
            pl.BlockSpec((_G, n, din), lambda i: (i, 0, 0)),
            full((din, dh)),
            full((1, dh)), full((1, dh)), full((1, dh)),
            full((1, dh)), full((1, dh)),
            full((1, dh)),
            full((1, 1)), full((1, 1)), full((1, 1)),
        ],
        out_specs=pl.BlockSpec((_G, n), lambda i: (i, 0)),
        out_shape=jax.ShapeDtypeStruct((b, n), jnp.float32),
    )(x.astype(jnp.bfloat16), W1,
      a1_src.reshape(1, dh), a1_dst.reshape(1, dh), b1.reshape(1, dh),
      gamma.reshape(1, dh), beta.reshape(1, dh),
      W2.reshape(1, dh),
      a2_src.reshape(1, 1), a2_dst.reshape(1, 1), b2.reshape(1, 1))
    return out.reshape(b, n, 1)


# speedup vs baseline: 1.0752x; 1.0069x over previous
import jax
import jax.numpy as jnp
from jax.experimental import pallas as pl

_G = 64


def _lrelu(v):
    return jnp.maximum(v, 0.2 * v)


def _gat_fused(x_ref, w1_ref, a1s_ref, a1d_ref, b1_ref, gamma_ref, beta_ref,
               w2_ref, a2s_ref, a2d_ref, b2_ref, out_ref):
    g, n, din = x_ref.shape
    dh = w1_ref.shape[1]

    x = x_ref[...].astype(jnp.float32).reshape(g * n, din)
    h_flat = jnp.dot(x, w1_ref[...], preferred_element_type=jnp.float32)
    h = h_flat.reshape(g, n, dh)

    s = (h_flat * a1s_ref[...]).sum(axis=1, keepdims=True).reshape(g, n, 1)
    d = (h_flat * a1d_ref[...]).sum(axis=1, keepdims=True).reshape(g, n, 1)
    d_row = jnp.swapaxes(d, 1, 2)
    dmax = jnp.max(d_row, axis=2, keepdims=True)
    m = _lrelu(s + dmax)
    p = jnp.exp(_lrelu(s + d_row) - m)
    ones = jnp.ones((g, n, 1), jnp.float32)
    ho = jnp.concatenate([h, ones], axis=2)
    acc = jax.lax.dot_general(p, ho,
                              (((2,), (1,)), ((0,), (0,))),
                              preferred_element_type=jnp.float32)
    out1 = acc[:, :, :dh] / acc[:, :, dh:] + b1_ref[...]

    mu = out1.mean(axis=2, keepdims=True)
    cent = out1 - mu
    var = (cent * cent).mean(axis=2, keepdims=True)
    xn = cent * jax.lax.rsqrt(var + 1e-5) * gamma_ref[...] + beta_ref[...]

    h2 = (xn * w2_ref[...]).sum(axis=2, keepdims=True)
    sa = a2s_ref[0, 0] * h2
    da = a2d_ref[0, 0] * h2
    da_row = jnp.swapaxes(da, 1, 2)
    m2 = _lrelu(sa + jnp.max(da_row, axis=2, keepdims=True))
    p2 = jnp.exp(_lrelu(sa + da_row) - m2)
    h2o = jnp.concatenate([h2, ones], axis=2)
    acc2 = jax.lax.dot_general(p2, h2o,
                               (((2,), (1,)), ((0,), (0,))),
                               preferred_element_type=jnp.float32)
    out2 = acc2[:, :, 0] / acc2[:, :, 1] + b2_ref[0, 0]
    out_ref[...] = jnp.where(out2 > 0.0, out2, jnp.exp(out2) - 1.0)


def kernel(x, adj, W1, a1_src, a1_dst, b1, gamma, beta, W2, a2_src, a2_dst, b2):
    del adj
    b, n, din = x.shape
    dh = W1.shape[1]
    full = lambda shape: pl.BlockSpec(shape, lambda i: (0, 0))
    out = pl.pallas_call(
        _gat_fused,
        grid=(b // _G,),
        in_specs=[
            pl.BlockSpec((_G, n, din), lambda i: (i, 0, 0)),
            full((din, dh)),
            full((1, dh)), full((1, dh)), full((1, dh)),
            full((1, dh)), full((1, dh)),
            full((1, dh)),
            full((1, 1)), full((1, 1)), full((1, 1)),
        ],
        out_specs=pl.BlockSpec((_G, n), lambda i: (i, 0)),
        out_shape=jax.ShapeDtypeStruct((b, n), jnp.float32),
    )(x.astype(jnp.bfloat16), W1,
      a1_src.reshape(1, dh), a1_dst.reshape(1, dh), b1.reshape(1, dh),
      gamma.reshape(1, dh), beta.reshape(1, dh),
      W2.reshape(1, dh),
      a2_src.reshape(1, 1), a2_dst.reshape(1, 1), b2.reshape(1, 1))
    return out.reshape(b, n, 1)
